# Initial kernel scaffold; baseline (speedup 1.0000x reference)
import jax
import jax.numpy as jnp
from jax.experimental import pallas as pl


def kernel(x, edge_index, batch, W_phi, b_phi, W_rho1, b_rho1, W_rho2, b_rho2):
    raise NotImplementedError("write your pallas kernel here")


# speedup vs baseline: 1.5983x; 1.5983x over previous
import functools

import jax
import jax.numpy as jnp
from jax import lax
from jax.experimental import pallas as pl
from jax.experimental.pallas import tpu as pltpu
from jax.experimental.pallas import tpu_sc as plsc

N_NODES = 10000
N_FEAT = 256
N_GRAPHS = 512

NG = 8
CG = 4
COLS = N_FEAT // CG
N_PAD = 10240
ROWS_PER_G = N_PAD // NG
RCHUNK = 256
NCHUNK = ROWS_PER_G // RCHUNK
G_ROWS = 640

PHI_BLOCK = 1024


def _phi_body(x_ref, w_ref, b_ref, o_ref):
    y = jnp.dot(x_ref[...], w_ref[...], preferred_element_type=jnp.float32)
    o_ref[...] = jnp.maximum(y + b_ref[...], 0.0)


def _phi(x, wt, b):
    grid = N_PAD // PHI_BLOCK
    return pl.pallas_call(
        _phi_body,
        grid=(grid,),
        in_specs=[
            pl.BlockSpec((PHI_BLOCK, N_FEAT), lambda i: (i, 0)),
            pl.BlockSpec((N_FEAT, N_FEAT), lambda i: (0, 0)),
            pl.BlockSpec((1, N_FEAT), lambda i: (0, 0)),
        ],
        out_specs=pl.BlockSpec((PHI_BLOCK, N_FEAT), lambda i: (i, 0)),
        out_shape=jax.ShapeDtypeStruct((N_PAD, N_FEAT), jnp.float32),
    )(x, wt, b)


_SC_MESH = plsc.VectorSubcoreMesh(core_axis_name="c", subcore_axis_name="s")


@functools.partial(
    pl.kernel,
    out_type=jax.ShapeDtypeStruct((NG, N_GRAPHS, N_FEAT), jnp.float32),
    mesh=_SC_MESH,
    compiler_params=pltpu.CompilerParams(
        use_tc_tiling_on_sc=False, needs_layout_passes=False
    ),
    scratch_types=[
        pltpu.VMEM((ROWS_PER_G,), jnp.int32),
        pltpu.VMEM((RCHUNK, COLS), jnp.float32),
        pltpu.VMEM((G_ROWS, COLS), jnp.float32),
    ],
)
def _pool(e_hbm, b_hbm, out_hbm, idx_v, rows_v, acc_v):
    cid = lax.axis_index("c")
    sid = lax.axis_index("s")
    wid = sid * 2 + cid
    ng = wid // CG
    cg = wid % CG
    col0 = cg * COLS
    row0 = ng * ROWS_PER_G

    zeros16 = jnp.zeros((16,), jnp.float32)

    def _zero(i, carry):
        for j in range(COLS // 16):
            acc_v[i, pl.ds(j * 16, 16)] = zeros16
        return carry

    lax.fori_loop(0, G_ROWS, _zero, 0)

    pltpu.sync_copy(b_hbm.at[pl.ds(row0, ROWS_PER_G)], idx_v)

    iotas = [lax.iota(jnp.int32, 16) + j * 16 for j in range(COLS // 16)]

    def _chunk(k, carry):
        pltpu.sync_copy(
            e_hbm.at[pl.ds(row0 + k * RCHUNK, RCHUNK), pl.ds(col0, COLS)],
            rows_v,
        )

        def _rows(g, carry2):
            base = g * 16
            idxv = idx_v[pl.ds(k * RCHUNK + base, 16)]
            for l in range(16):
                bvec = jnp.full((16,), idxv[l], dtype=jnp.int32)
                for j in range(COLS // 16):
                    v = rows_v[base + l, pl.ds(j * 16, 16)]
                    plsc.addupdate_scatter(acc_v, [bvec, iotas[j]], v)
            return carry2

        lax.fori_loop(0, RCHUNK // 16, _rows, 0)
        return carry

    lax.fori_loop(0, NCHUNK, _chunk, 0)

    pltpu.sync_copy(
        acc_v.at[pl.ds(0, N_GRAPHS)],
        out_hbm.at[ng, pl.ds(0, N_GRAPHS), pl.ds(col0, COLS)],
    )


def _rho_body(g_ref, w1_ref, b1_ref, w2_ref, b2_ref, o_ref):
    g = g_ref[0]
    for p in range(1, NG):
        g = g + g_ref[p]
    h = jnp.dot(g, w1_ref[...], preferred_element_type=jnp.float32)
    h = jnp.maximum(h + b1_ref[...], 0.0)
    y = jnp.dot(h, w2_ref[...], preferred_element_type=jnp.float32)
    o_ref[...] = jax.nn.sigmoid(y + b2_ref[...])


def _rho(gp, w1t, b1, w2p, b2p):
    return pl.pallas_call(
        _rho_body,
        out_shape=jax.ShapeDtypeStruct((N_GRAPHS, 128), jnp.float32),
    )(gp, w1t, b1, w2p, b2p)


def kernel(x, edge_index, batch, W_phi, b_phi, W_rho1, b_rho1, W_rho2, b_rho2):
    del edge_index
    batch_pad = jnp.concatenate(
        [batch.astype(jnp.int32),
         jnp.full((N_PAD - N_NODES,), N_GRAPHS, dtype=jnp.int32)]
    )

    e = _phi(x, W_phi.T, b_phi.reshape(1, N_FEAT))
    gp = _pool(e, batch_pad)

    w2p = jnp.pad(W_rho2.T, ((0, 0), (0, 127)))
    b2p = jnp.pad(b_rho2, (0, 127))
    out = _rho(gp, W_rho1.T, b_rho1.reshape(1, N_FEAT), w2p, b2p.reshape(1, 128))
    return out[:, 0]


# speedup vs baseline: 2.2190x; 1.3884x over previous
import functools

import jax
import jax.numpy as jnp
from jax import lax
from jax.experimental import pallas as pl
from jax.experimental.pallas import tpu as pltpu
from jax.experimental.pallas import tpu_sc as plsc

N_NODES = 10000
N_FEAT = 256
N_GRAPHS = 512

NG = 16
CG = 2
COLS = N_FEAT // CG
N_PAD = 10240
ROWS_PER_G = N_PAD // NG
RCHUNK = 160
NCHUNK = ROWS_PER_G // RCHUNK
G_ROWS = 520

PHI_BLOCK = 1024


def _phi_body(x_ref, w_ref, b_ref, o_ref):
    y = jnp.dot(x_ref[...], w_ref[...], preferred_element_type=jnp.float32)
    o_ref[...] = jnp.maximum(y + b_ref[...], 0.0)[None]


def _phi(x, wt, b):
    return pl.pallas_call(
        _phi_body,
        grid=(N_PAD // PHI_BLOCK, CG),
        in_specs=[
            pl.BlockSpec((PHI_BLOCK, N_FEAT), lambda i, j: (i, 0)),
            pl.BlockSpec((N_FEAT, COLS), lambda i, j: (0, j)),
            pl.BlockSpec((1, COLS), lambda i, j: (0, j)),
        ],
        out_specs=pl.BlockSpec((1, PHI_BLOCK, COLS), lambda i, j: (j, i, 0)),
        out_shape=jax.ShapeDtypeStruct((CG, N_PAD, COLS), jnp.float32),
    )(x, wt, b)


_SC_MESH = plsc.VectorSubcoreMesh(core_axis_name="c", subcore_axis_name="s")


@functools.partial(
    pl.kernel,
    out_type=jax.ShapeDtypeStruct((NG, N_GRAPHS, N_FEAT), jnp.float32),
    mesh=_SC_MESH,
    compiler_params=pltpu.CompilerParams(needs_layout_passes=False),
    scratch_types=[
        pltpu.VMEM((ROWS_PER_G,), jnp.int32),
        pltpu.VMEM((2, RCHUNK, COLS), jnp.float32),
        pltpu.VMEM((G_ROWS, COLS), jnp.float32),
        pltpu.SemaphoreType.DMA,
        pltpu.SemaphoreType.DMA,
        pltpu.SemaphoreType.DMA,
    ],
)
def _pool(e_hbm, b_hbm, out_hbm, idx_v, rows_v, acc_v, sem_i, sem_a, sem_b):
    cid = lax.axis_index("c")
    sid = lax.axis_index("s")
    row0 = sid * ROWS_PER_G

    idx_cp = pltpu.async_copy(
        b_hbm.at[pl.ds(row0, ROWS_PER_G)], idx_v, sem_i
    )
    sems = (sem_a, sem_b)

    def _start(k):
        return pltpu.async_copy(
            e_hbm.at[cid, pl.ds(row0 + k * RCHUNK, RCHUNK)],
            rows_v.at[k % 2],
            sems[k % 2],
        )

    cp = _start(0)

    zeros16 = jnp.zeros((16,), jnp.float32)

    def _zero(i, carry):
        for j in range(COLS // 16):
            acc_v[i, pl.ds(j * 16, 16)] = zeros16
        return carry

    lax.fori_loop(0, G_ROWS, _zero, 0)
    idx_cp.wait()

    iotas = [lax.iota(jnp.int32, 16) + j * 16 for j in range(COLS // 16)]

    for k in range(NCHUNK):
        cp.wait()
        if k + 1 < NCHUNK:
            cp = _start(k + 1)
        buf = k % 2

        def _rows(g, carry, _k=k, _buf=buf):
            base = g * 16
            idxv = idx_v[pl.ds(_k * RCHUNK + base, 16)]
            for l in range(16):
                bvec = jnp.full((16,), idxv[l], dtype=jnp.int32)
                vals = [
                    rows_v[_buf, base + l, pl.ds(j * 16, 16)]
                    for j in range(COLS // 16)
                ]
                for j in range(COLS // 16):
                    plsc.addupdate_scatter(acc_v, [bvec, iotas[j]], vals[j])
            return carry

        lax.fori_loop(0, RCHUNK // 16, _rows, 0)

    pltpu.sync_copy(
        acc_v.at[pl.ds(0, N_GRAPHS)],
        out_hbm.at[sid, pl.ds(0, N_GRAPHS), pl.ds(cid * COLS, COLS)],
    )


def _rho_body(g_ref, w1_ref, b1_ref, w2_ref, b2_ref, o_ref):
    g = g_ref[0]
    for p in range(1, NG):
        g = g + g_ref[p]
    h = jnp.dot(g, w1_ref[...], preferred_element_type=jnp.float32)
    h = jnp.maximum(h + b1_ref[...], 0.0)
    y = jnp.dot(h, w2_ref[...], preferred_element_type=jnp.float32)
    o_ref[...] = jax.nn.sigmoid(y + b2_ref[...])


def _rho(gp, w1t, b1, w2p, b2p):
    return pl.pallas_call(
        _rho_body,
        out_shape=jax.ShapeDtypeStruct((N_GRAPHS, 128), jnp.float32),
    )(gp, w1t, b1, w2p, b2p)


def kernel(x, edge_index, batch, W_phi, b_phi, W_rho1, b_rho1, W_rho2, b_rho2):
    del edge_index
    batch_pad = jnp.concatenate(
        [batch.astype(jnp.int32),
         jnp.full((N_PAD - N_NODES,), N_GRAPHS, dtype=jnp.int32)]
    )

    e = _phi(x, W_phi.T, b_phi.reshape(1, N_FEAT))
    gp = _pool(e, batch_pad)

    w2p = jnp.pad(W_rho2.T, ((0, 0), (0, 127)))
    b2p = jnp.pad(b_rho2, (0, 127))
    out = _rho(gp, W_rho1.T, b_rho1.reshape(1, N_FEAT), w2p, b2p.reshape(1, 128))
    return out[:, 0]


# speedup vs baseline: 2.6806x; 1.2080x over previous
import functools

import jax
import jax.numpy as jnp
from jax import lax
from jax.experimental import pallas as pl
from jax.experimental.pallas import tpu as pltpu
from jax.experimental.pallas import tpu_sc as plsc

N_NODES = 10000
N_FEAT = 256
N_GRAPHS = 512

CG = 2
COLS = N_FEAT // CG
GPERS = N_GRAPHS // 16
RCHUNK = 160
N_PAD = 10400
G_ROWS = 520
SEARCH_ITERS = 14

PHI_BLOCK = 1040


def _phi_body(x_ref, w_ref, b_ref, o_ref):
    y = jnp.dot(x_ref[...], w_ref[...], preferred_element_type=jnp.float32)
    y = jnp.maximum(y + b_ref[...], 0.0)
    o_ref[0] = y[:, :COLS]
    o_ref[1] = y[:, COLS:]


def _phi(x, wt, b):
    return pl.pallas_call(
        _phi_body,
        grid=(N_PAD // PHI_BLOCK,),
        in_specs=[
            pl.BlockSpec((PHI_BLOCK, N_FEAT), lambda i: (i, 0)),
            pl.BlockSpec((N_FEAT, N_FEAT), lambda i: (0, 0)),
            pl.BlockSpec((1, N_FEAT), lambda i: (0, 0)),
        ],
        out_specs=pl.BlockSpec((CG, PHI_BLOCK, COLS), lambda i: (0, i, 0)),
        out_shape=jax.ShapeDtypeStruct((CG, N_PAD, COLS), jnp.float32),
    )(x, wt, b)


_SC_MESH = plsc.VectorSubcoreMesh(core_axis_name="c", subcore_axis_name="s")


@functools.partial(
    pl.kernel,
    out_type=jax.ShapeDtypeStruct((N_GRAPHS, N_FEAT), jnp.float32),
    mesh=_SC_MESH,
    compiler_params=pltpu.CompilerParams(needs_layout_passes=False),
    scratch_types=[
        pltpu.VMEM((N_PAD,), jnp.int32),
        pltpu.VMEM((2, RCHUNK, COLS), jnp.float32),
        pltpu.VMEM((G_ROWS, COLS), jnp.float32),
        pltpu.SemaphoreType.DMA,
        pltpu.SemaphoreType.DMA,
    ],
)
def _pool(e_hbm, b_hbm, out_hbm, idx_v, rows_v, acc_v, sem_i, sem_r):
    cid = lax.axis_index("c")
    sid = lax.axis_index("s")
    g_lo = sid * GPERS

    idx_cp = pltpu.async_copy(b_hbm, idx_v, sem_i)

    zeros16 = jnp.zeros((16,), jnp.float32)

    def _zero(i, carry):
        for j in range(COLS // 16):
            acc_v[g_lo + i, pl.ds(j * 16, 16)] = zeros16
        return carry

    lax.fori_loop(0, GPERS, _zero, 0)
    idx_cp.wait()

    def _lower_bound(target):
        def body(_, lohi):
            lo, hi = lohi
            mid = (lo + hi) // 2
            v = idx_v[pl.ds(mid, 16)][0]
            lt = v < target
            return (jnp.where(lt, mid + 1, lo), jnp.where(lt, hi, mid))

        lo, _ = lax.fori_loop(
            0, SEARCH_ITERS, body, (jnp.int32(0), jnp.int32(N_PAD))
        )
        return lo

    start = _lower_bound(g_lo)
    end = _lower_bound(g_lo + GPERS)
    start8 = (start // 8) * 8
    nch = (end - start8 + (RCHUNK - 1)) // RCHUNK

    def _row_copy(k, par):
        return pltpu.make_async_copy(
            e_hbm.at[cid, pl.ds(start8 + k * RCHUNK, RCHUNK)],
            rows_v.at[par],
            sem_r,
        )

    @pl.when(nch > 0)
    def _prologue():
        _row_copy(jnp.int32(0), jnp.int32(0)).start()

    iotas = [lax.iota(jnp.int32, 16) + j * 16 for j in range(COLS // 16)]

    def _chunk(k, carry):
        par = lax.rem(k, 2)

        @pl.when(k + 1 < nch)
        def _next():
            _row_copy(k + 1, lax.rem(k + 1, 2)).start()

        _row_copy(k, par).wait()
        off = start8 + k * RCHUNK

        def _rows(g, carry2):
            base = g * 16
            idxv = idx_v[pl.ds(off + base, 16)]
            for l in range(16):
                bvec = jnp.full((16,), idxv[l], dtype=jnp.int32)
                vals = [
                    rows_v[par, base + l, pl.ds(j * 16, 16)]
                    for j in range(COLS // 16)
                ]
                for j in range(COLS // 16):
                    plsc.addupdate_scatter(acc_v, [bvec, iotas[j]], vals[j])
            return carry2

        lax.fori_loop(0, RCHUNK // 16, _rows, 0)
        return carry

    lax.fori_loop(0, nch, _chunk, 0)

    pltpu.sync_copy(
        acc_v.at[pl.ds(g_lo, GPERS)],
        out_hbm.at[pl.ds(g_lo, GPERS), pl.ds(cid * COLS, COLS)],
    )


def _rho_body(g_ref, w1_ref, b1_ref, w2_ref, b2_ref, o_ref):
    h = jnp.dot(g_ref[...], w1_ref[...], preferred_element_type=jnp.float32)
    h = jnp.maximum(h + b1_ref[...], 0.0)
    y = jnp.dot(h, w2_ref[...], preferred_element_type=jnp.float32)
    o_ref[...] = jax.nn.sigmoid(y + b2_ref[...])[:, :1]


def _rho(g, w1t, b1, w2p, b2p):
    return pl.pallas_call(
        _rho_body,
        out_shape=jax.ShapeDtypeStruct((N_GRAPHS, 1), jnp.float32),
    )(g, w1t, b1, w2p, b2p)


def kernel(x, edge_index, batch, W_phi, b_phi, W_rho1, b_rho1, W_rho2, b_rho2):
    del edge_index
    batch_pad = jnp.concatenate(
        [batch.astype(jnp.int32),
         jnp.full((N_PAD - N_NODES,), N_GRAPHS, dtype=jnp.int32)]
    )

    e = _phi(x, W_phi.T, b_phi.reshape(1, N_FEAT))
    g = _pool(e, batch_pad)

    w2p = jnp.pad(W_rho2.T, ((0, 0), (0, 127)))
    b2p = jnp.pad(b_rho2, (0, 127))
    out = _rho(g, W_rho1.T, b_rho1.reshape(1, N_FEAT), w2p, b2p.reshape(1, 128))
    return out.reshape(N_GRAPHS)


# speedup vs baseline: 2.8864x; 1.0768x over previous
import functools

import jax
import jax.numpy as jnp
from jax import lax
from jax.experimental import pallas as pl
from jax.experimental.pallas import tpu as pltpu
from jax.experimental.pallas import tpu_sc as plsc

N_NODES = 10000
N_FEAT = 256
N_GRAPHS = 512

CG = 2
COLS = N_FEAT // CG
GPERS = N_GRAPHS // 16
RCHUNK = 160
N_PAD = 10400
G_ROWS = 520
SEARCH_ITERS = 14

PHI_BLOCK = 1040


def _phi_body(x_ref, w_ref, b_ref, o_ref):
    y = lax.dot_general(
        x_ref[...], w_ref[...], (((1,), (1,)), ((), ())),
        preferred_element_type=jnp.float32,
    )
    y = jnp.maximum(y + b_ref[...], 0.0)
    o_ref[0] = y[:, :COLS]
    o_ref[1] = y[:, COLS:]


def _phi(x, wt, b):
    return pl.pallas_call(
        _phi_body,
        grid=(N_PAD // PHI_BLOCK,),
        in_specs=[
            pl.BlockSpec((PHI_BLOCK, N_FEAT), lambda i: (i, 0)),
            pl.BlockSpec((N_FEAT, N_FEAT), lambda i: (0, 0)),
            pl.BlockSpec((1, N_FEAT), lambda i: (0, 0)),
        ],
        out_specs=pl.BlockSpec((CG, PHI_BLOCK, COLS), lambda i: (0, i, 0)),
        out_shape=jax.ShapeDtypeStruct((CG, N_PAD, COLS), jnp.float32),
    )(x, wt, b)


_SC_MESH = plsc.VectorSubcoreMesh(core_axis_name="c", subcore_axis_name="s")


@functools.partial(
    pl.kernel,
    out_type=jax.ShapeDtypeStruct((N_GRAPHS, N_FEAT), jnp.float32),
    mesh=_SC_MESH,
    compiler_params=pltpu.CompilerParams(needs_layout_passes=False),
    scratch_types=[
        pltpu.VMEM((N_PAD,), jnp.int32),
        pltpu.VMEM((2, RCHUNK, COLS), jnp.float32),
        pltpu.VMEM((G_ROWS, COLS), jnp.float32),
        pltpu.SemaphoreType.DMA,
        pltpu.SemaphoreType.DMA,
    ],
)
def _pool(e_hbm, b_hbm, out_hbm, idx_v, rows_v, acc_v, sem_i, sem_r):
    cid = lax.axis_index("c")
    sid = lax.axis_index("s")
    g_lo = sid * GPERS

    idx_cp = pltpu.async_copy(b_hbm, idx_v, sem_i)

    zeros16 = jnp.zeros((16,), jnp.float32)

    def _zero(i, carry):
        for j in range(COLS // 16):
            acc_v[g_lo + i, pl.ds(j * 16, 16)] = zeros16
        return carry

    lax.fori_loop(0, GPERS, _zero, 0)
    idx_cp.wait()

    def _lower_bound(target):
        def body(_, lohi):
            lo, hi = lohi
            mid = (lo + hi) // 2
            v = idx_v[pl.ds(mid, 16)][0]
            lt = v < target
            return (jnp.where(lt, mid + 1, lo), jnp.where(lt, hi, mid))

        lo, _ = lax.fori_loop(
            0, SEARCH_ITERS, body, (jnp.int32(0), jnp.int32(N_PAD))
        )
        return lo

    start = _lower_bound(g_lo)
    end = _lower_bound(g_lo + GPERS)
    start8 = (start // 8) * 8
    nch = (end - start8 + (RCHUNK - 1)) // RCHUNK

    def _row_copy(k, par):
        return pltpu.make_async_copy(
            e_hbm.at[cid, pl.ds(start8 + k * RCHUNK, RCHUNK)],
            rows_v.at[par],
            sem_r,
        )

    @pl.when(nch > 0)
    def _prologue():
        _row_copy(jnp.int32(0), jnp.int32(0)).start()

    iotas = [lax.iota(jnp.int32, 16) + j * 16 for j in range(COLS // 16)]

    def _chunk(k, carry):
        par = lax.rem(k, 2)

        @pl.when(k + 1 < nch)
        def _next():
            _row_copy(k + 1, lax.rem(k + 1, 2)).start()

        _row_copy(k, par).wait()
        off = start8 + k * RCHUNK

        def _rows(g, carry2):
            base = g * 16
            idxv = idx_v[pl.ds(off + base, 16)]
            for l in range(16):
                bvec = jnp.full((16,), idxv[l], dtype=jnp.int32)
                vals = [
                    rows_v[par, base + l, pl.ds(j * 16, 16)]
                    for j in range(COLS // 16)
                ]
                for j in range(COLS // 16):
                    plsc.addupdate_scatter(acc_v, [bvec, iotas[j]], vals[j])
            return carry2

        lax.fori_loop(0, RCHUNK // 16, _rows, 0)
        return carry

    lax.fori_loop(0, nch, _chunk, 0)

    pltpu.sync_copy(
        acc_v.at[pl.ds(g_lo, GPERS)],
        out_hbm.at[pl.ds(g_lo, GPERS), pl.ds(cid * COLS, COLS)],
    )


def _rho_body(g_ref, w1_ref, b1_ref, w2_ref, b2_ref, o_ref):
    h = lax.dot_general(
        g_ref[...], w1_ref[...], (((1,), (1,)), ((), ())),
        preferred_element_type=jnp.float32,
    )
    h = jnp.maximum(h + b1_ref[...], 0.0)
    y = lax.dot_general(
        w2_ref[...], h, (((1,), (1,)), ((), ())),
        preferred_element_type=jnp.float32,
    )
    o_ref[...] = jax.nn.sigmoid(y + b2_ref[...])


def _rho(g, w1, b1, w2, b2):
    return pl.pallas_call(
        _rho_body,
        out_shape=jax.ShapeDtypeStruct((1, N_GRAPHS), jnp.float32),
    )(g, w1, b1, w2, b2)


def kernel(x, edge_index, batch, W_phi, b_phi, W_rho1, b_rho1, W_rho2, b_rho2):
    del edge_index
    batch_pad = jnp.concatenate(
        [batch.astype(jnp.int32),
         jnp.full((N_PAD - N_NODES,), N_GRAPHS, dtype=jnp.int32)]
    )

    e = _phi(x, W_phi, b_phi.reshape(1, N_FEAT))
    g = _pool(e, batch_pad)

    out = _rho(g, W_rho1, b_rho1.reshape(1, N_FEAT), W_rho2, b_rho2.reshape(1, 1))
    return out.reshape(N_GRAPHS)


# speedup vs baseline: 3.0534x; 1.0579x over previous
import functools

import jax
import jax.numpy as jnp
from jax import lax
from jax.experimental import pallas as pl
from jax.experimental.pallas import tpu as pltpu
from jax.experimental.pallas import tpu_sc as plsc

N_NODES = 10000
N_FEAT = 256
N_GRAPHS = 512

NW = 32
GPERS = N_GRAPHS // NW
WORDS = N_FEAT // 2
RCHUNK = 160
N_PAD = 10400
SEARCH_ITERS = 14

PHI_BLOCK = 1040


def _phi_body(x_ref, w_ref, b_ref, o_ref):
    y = lax.dot_general(
        x_ref[...], w_ref[...], (((1,), (1,)), ((), ())),
        preferred_element_type=jnp.float32,
    )
    y = jnp.maximum(y + b_ref[...], 0.0)
    bits = lax.bitcast_convert_type(y, jnp.int32) + 0x8000
    lo = lax.shift_right_logical(bits[:, :WORDS], 16)
    hi = lax.bitwise_and(bits[:, WORDS:], -65536)
    o_ref[...] = lax.bitwise_or(hi, lo)


def _phi(x, w, b):
    return pl.pallas_call(
        _phi_body,
        grid=(N_PAD // PHI_BLOCK,),
        in_specs=[
            pl.BlockSpec((PHI_BLOCK, N_FEAT), lambda i: (i, 0)),
            pl.BlockSpec((N_FEAT, N_FEAT), lambda i: (0, 0)),
            pl.BlockSpec((1, N_FEAT), lambda i: (0, 0)),
        ],
        out_specs=pl.BlockSpec((PHI_BLOCK, WORDS), lambda i: (i, 0)),
        out_shape=jax.ShapeDtypeStruct((N_PAD, WORDS), jnp.int32),
    )(x, w, b)


_SC_MESH = plsc.VectorSubcoreMesh(core_axis_name="c", subcore_axis_name="s")


@functools.partial(
    pl.kernel,
    out_type=jax.ShapeDtypeStruct((N_GRAPHS, N_FEAT), jnp.float32),
    mesh=_SC_MESH,
    compiler_params=pltpu.CompilerParams(needs_layout_passes=False),
    scratch_types=[
        pltpu.VMEM((N_PAD,), jnp.int32),
        pltpu.VMEM((2, RCHUNK, WORDS), jnp.int32),
        pltpu.VMEM((GPERS, N_FEAT), jnp.float32),
        pltpu.SemaphoreType.DMA,
        pltpu.SemaphoreType.DMA,
    ],
)
def _pool(e_hbm, b_hbm, out_hbm, idx_v, rows_v, acc_v, sem_i, sem_r):
    cid = lax.axis_index("c")
    sid = lax.axis_index("s")
    wid = sid * 2 + cid
    g_lo = wid * GPERS
    g_hi = g_lo + GPERS

    idx_cp = pltpu.async_copy(b_hbm, idx_v, sem_i)

    zeros16 = jnp.zeros((16,), jnp.float32)

    def _zero(i, carry):
        for j in range(N_FEAT // 16):
            acc_v[i, pl.ds(j * 16, 16)] = zeros16
        return carry

    lax.fori_loop(0, GPERS, _zero, 0)
    idx_cp.wait()

    def _lower_bound(target):
        def body(_, lohi):
            lo, hi = lohi
            mid = (lo + hi) // 2
            v = idx_v[pl.ds(mid, 16)][0]
            lt = v < target
            return (jnp.where(lt, mid + 1, lo), jnp.where(lt, hi, mid))

        lo, _ = lax.fori_loop(
            0, SEARCH_ITERS, body, (jnp.int32(0), jnp.int32(N_PAD))
        )
        return lo

    start = _lower_bound(g_lo)
    end = _lower_bound(g_hi)
    start8 = (start // 8) * 8
    nch = (end - start8 + (RCHUNK - 1)) // RCHUNK

    def _row_copy(k, par):
        return pltpu.make_async_copy(
            e_hbm.at[pl.ds(start8 + k * RCHUNK, RCHUNK)],
            rows_v.at[par],
            sem_r,
        )

    @pl.when(nch > 0)
    def _prologue():
        _row_copy(jnp.int32(0), jnp.int32(0)).start()

    iota16 = lax.iota(jnp.int32, 16)
    iotas = []
    for j in range(WORDS // 16):
        iotas.append(iota16 + 16 * j)
        iotas.append(iota16 + 16 * j + WORDS)

    himask = jnp.full((16,), -65536, dtype=jnp.int32)

    def _chunk(k, carry):
        par = lax.rem(k, 2)

        @pl.when(k + 1 < nch)
        def _next():
            _row_copy(k + 1, lax.rem(k + 1, 2)).start()

        _row_copy(k, par).wait()
        off = start8 + k * RCHUNK

        def _rows(g, carry2):
            base = g * 16
            idxv = idx_v[pl.ds(off + base, 16)]

            def _load(l):
                vs = []
                for j in range(WORDS // 16):
                    w = rows_v[par, base + l, pl.ds(j * 16, 16)]
                    vs.append(plsc.bitcast(lax.shift_left(w, 16), jnp.float32))
                    vs.append(plsc.bitcast(lax.bitwise_and(w, himask), jnp.float32))
                return vs

            vals = _load(0)
            for l in range(16):
                nxt = _load(l + 1) if l < 15 else None
                bvec = jnp.full((16,), idxv[l], dtype=jnp.int32)
                bl = bvec - g_lo
                m = jnp.logical_and(bvec >= g_lo, bvec < g_hi)
                for j in range(N_FEAT // 16):
                    plsc.addupdate_scatter(acc_v, [bl, iotas[j]], vals[j], mask=m)
                vals = nxt
            return carry2

        lax.fori_loop(0, RCHUNK // 16, _rows, 0)
        return carry

    lax.fori_loop(0, nch, _chunk, 0)

    pltpu.sync_copy(acc_v, out_hbm.at[pl.ds(g_lo, GPERS)])


def _rho_body(g_ref, w1_ref, b1_ref, w2_ref, b2_ref, o_ref):
    h = lax.dot_general(
        g_ref[...], w1_ref[...], (((1,), (1,)), ((), ())),
        preferred_element_type=jnp.float32,
    )
    h = jnp.maximum(h + b1_ref[...], 0.0)
    y = lax.dot_general(
        w2_ref[...], h, (((1,), (1,)), ((), ())),
        preferred_element_type=jnp.float32,
    )
    o_ref[...] = jax.nn.sigmoid(y + b2_ref[...])


def _rho(g, w1, b1, w2, b2):
    return pl.pallas_call(
        _rho_body,
        out_shape=jax.ShapeDtypeStruct((1, N_GRAPHS), jnp.float32),
    )(g, w1, b1, w2, b2)


def kernel(x, edge_index, batch, W_phi, b_phi, W_rho1, b_rho1, W_rho2, b_rho2):
    del edge_index
    batch_pad = jnp.concatenate(
        [batch.astype(jnp.int32),
         jnp.full((N_PAD - N_NODES,), N_GRAPHS, dtype=jnp.int32)]
    )

    e = _phi(x, W_phi, b_phi.reshape(1, N_FEAT))
    g = _pool(e, batch_pad)

    out = _rho(g, W_rho1, b_rho1.reshape(1, N_FEAT), W_rho2, b_rho2.reshape(1, 1))
    return out.reshape(N_GRAPHS)
